# baseline (speedup 1.0000x reference)
import functools

import jax
import jax.numpy as jnp
from jax import lax
from jax.experimental import pallas as pl
from jax.experimental.pallas import tpu as pltpu

_NUM_TOKENS = 4096
_MODEL_DIM = 4096
_NUM_EXPERTS = 64
_CAPACITY = 128

_B1 = 512
_NB1 = _NUM_TOKENS // _B1
_B2 = 256
_NB2 = _NUM_TOKENS // _B2
_B3 = 256
_NB3 = _NUM_TOKENS // _B3

_GUMBEL_CACHE = None


def _gumbel_const():
    global _GUMBEL_CACHE
    if _GUMBEL_CACHE is None:
        _GUMBEL_CACHE = jax.random.gumbel(
            jax.random.key(42), (_NUM_TOKENS, _NUM_EXPERTS), jnp.float32)
    return _GUMBEL_CACHE


def _phase1_body(x_ref, wg_ref, gum_ref,
                 l1_ref, l2_ref, i1_ref, i2_ref, lc1_ref, lc2_ref,
                 cnt_ref, laux_ref,
                 accg_ref, c1_ref, c2_ref):
    i = pl.program_id(0)

    @pl.when(i == 0)
    def _init():
        accg_ref[...] = jnp.zeros_like(accg_ref)
        c1_ref[...] = jnp.zeros_like(c1_ref)
        c2_ref[...] = jnp.zeros_like(c2_ref)

    logits = jnp.dot(x_ref[...], wg_ref[...],
                     preferred_element_type=jnp.float32)
    iota_e = jax.lax.broadcasted_iota(jnp.int32, (_B1, _NUM_EXPERTS), 1)

    m1 = jnp.max(logits, axis=1, keepdims=True)
    idx1 = jnp.min(jnp.where(logits == m1, iota_e, _NUM_EXPERTS),
                   axis=1, keepdims=True)
    eq1 = iota_e == idx1

    e = jnp.exp(logits - m1)
    gates = e / jnp.sum(e, axis=1, keepdims=True)
    accg_ref[...] += jnp.sum(gates, axis=0, keepdims=True)

    lw = logits + gum_ref[...]
    lw1 = jnp.where(eq1, -jnp.inf, lw)
    m2 = jnp.max(lw1, axis=1, keepdims=True)
    idx2 = jnp.min(jnp.where(lw1 == m2, iota_e, _NUM_EXPERTS),
                   axis=1, keepdims=True)
    eq2 = iota_e == idx2
    l2v = jnp.sum(jnp.where(eq2, logits, 0.0), axis=1, keepdims=True)

    m1f = eq1.astype(jnp.float32)
    m2f = eq2.astype(jnp.float32)
    r = jax.lax.broadcasted_iota(jnp.int32, (_B1, _B1), 0)
    c = jax.lax.broadcasted_iota(jnp.int32, (_B1, _B1), 1)
    tri = (r >= c).astype(jnp.float32)
    cum1 = jnp.dot(tri, m1f, preferred_element_type=jnp.float32)
    cum2 = jnp.dot(tri, m2f, preferred_element_type=jnp.float32)

    c1p = c1_ref[...].astype(jnp.float32)
    c2p = c2_ref[...].astype(jnp.float32)
    loc1 = jnp.sum((c1p + cum1 - 1.0) * m1f, axis=1,
                   keepdims=True).astype(jnp.int32)
    loc2 = jnp.sum((c2p + cum2 - 1.0) * m2f, axis=1,
                   keepdims=True).astype(jnp.int32)

    c1_ref[...] += jnp.sum(eq1.astype(jnp.int32), axis=0, keepdims=True)
    c2_ref[...] += jnp.sum(eq2.astype(jnp.int32), axis=0, keepdims=True)

    l1_ref[...] = m1
    l2_ref[...] = l2v
    i1_ref[...] = idx1
    i2_ref[...] = idx2
    lc1_ref[...] = loc1
    lc2_ref[...] = loc2

    @pl.when(i == _NB1 - 1)
    def _finalize():
        cnt = c1_ref[...]
        cnt_ref[...] = cnt
        me = accg_ref[...] / _NUM_TOKENS
        ce = cnt.astype(jnp.float32) / _NUM_TOKENS
        laux_ref[...] = (jnp.sum(me * ce) * _NUM_EXPERTS).reshape(1, 1)


def _phase2_body(l1_ref, l2_ref, i1_ref, i2_ref, lc1_ref, lc2_ref, cnt_ref,
                 outc_ref, pid_ref):
    l1 = l1_ref[...]
    l2 = l2_ref[...]
    idx1 = i1_ref[...]
    idx2 = i2_ref[...]
    loc1 = lc1_ref[...]
    cnt = cnt_ref[...]

    iota_e = jax.lax.broadcasted_iota(jnp.int32, (_B2, _NUM_EXPERTS), 1)
    eq1 = iota_e == idx1
    eq2 = iota_e == idx2
    add2 = jnp.sum(jnp.where(eq2, cnt, 0), axis=1, keepdims=True)
    loc2 = lc2_ref[...] + add2

    k1 = (loc1 < _CAPACITY).astype(jnp.float32)
    k2 = (loc2 < _CAPACITY).astype(jnp.float32)
    b = jnp.exp(l2 - l1)
    denom = k1 + k2 * b
    inv = jnp.where(denom > 0, 1.0 / jnp.where(denom > 0, denom, 1.0), 0.0)
    w1 = k1 * inv
    w2 = k2 * b * inv

    iota_c = jax.lax.broadcasted_iota(jnp.int32, (_B2, _CAPACITY), 1)
    oh_c1 = (iota_c == loc1).astype(jnp.float32)
    oh_c2 = (iota_c == loc2).astype(jnp.float32)

    t1 = (w1 * eq1.astype(jnp.float32))[:, :, None] * oh_c1[:, None, :]
    t2 = (w2 * eq2.astype(jnp.float32))[:, :, None] * oh_c2[:, None, :]
    outc_ref[...] = t1 + t2

    keep1 = jnp.logical_and(eq1, loc1 < _CAPACITY)
    keep2 = jnp.logical_and(eq2, loc2 < _CAPACITY)
    pid_ref[...] = jnp.where(keep1, loc1,
                             jnp.where(keep2, loc2, _CAPACITY))


def _phase3_body(pid_ref, outm_ref):
    iota_c = jax.lax.broadcasted_iota(
        jnp.int32, (_B3, _NUM_EXPERTS, _CAPACITY), 2)
    outm_ref[...] = iota_c == pid_ref[...][:, :, None]


def kernel(input, wg):
    gum = _gumbel_const()
    f32 = jnp.float32
    i32 = jnp.int32
    tok_spec = pl.BlockSpec((_B1, 1), lambda i: (i, 0))
    l1, l2, i1, i2, lc1, lc2, cnt, laux = pl.pallas_call(
        _phase1_body,
        grid=(_NB1,),
        in_specs=[
            pl.BlockSpec((_B1, _MODEL_DIM), lambda i: (i, 0)),
            pl.BlockSpec((_MODEL_DIM, _NUM_EXPERTS), lambda i: (0, 0)),
            pl.BlockSpec((_B1, _NUM_EXPERTS), lambda i: (i, 0)),
        ],
        out_specs=[
            tok_spec, tok_spec, tok_spec, tok_spec, tok_spec, tok_spec,
            pl.BlockSpec((1, _NUM_EXPERTS), lambda i: (0, 0)),
            pl.BlockSpec((1, 1), lambda i: (0, 0)),
        ],
        out_shape=[
            jax.ShapeDtypeStruct((_NUM_TOKENS, 1), f32),
            jax.ShapeDtypeStruct((_NUM_TOKENS, 1), f32),
            jax.ShapeDtypeStruct((_NUM_TOKENS, 1), i32),
            jax.ShapeDtypeStruct((_NUM_TOKENS, 1), i32),
            jax.ShapeDtypeStruct((_NUM_TOKENS, 1), i32),
            jax.ShapeDtypeStruct((_NUM_TOKENS, 1), i32),
            jax.ShapeDtypeStruct((1, _NUM_EXPERTS), i32),
            jax.ShapeDtypeStruct((1, 1), f32),
        ],
        scratch_shapes=[
            pltpu.VMEM((1, _NUM_EXPERTS), f32),
            pltpu.VMEM((1, _NUM_EXPERTS), i32),
            pltpu.VMEM((1, _NUM_EXPERTS), i32),
        ],
    )(input, wg, gum)

    tok2_spec = pl.BlockSpec((_B2, 1), lambda i: (i, 0))
    out3_spec = pl.BlockSpec((_B2, _NUM_EXPERTS, _CAPACITY),
                             lambda i: (i, 0, 0))
    outc, pid = pl.pallas_call(
        _phase2_body,
        grid=(_NB2,),
        in_specs=[
            tok2_spec, tok2_spec, tok2_spec, tok2_spec, tok2_spec, tok2_spec,
            pl.BlockSpec((1, _NUM_EXPERTS), lambda i: (0, 0)),
        ],
        out_specs=[out3_spec, pl.BlockSpec((_B2, _NUM_EXPERTS),
                                           lambda i: (i, 0))],
        out_shape=[
            jax.ShapeDtypeStruct((_NUM_TOKENS, _NUM_EXPERTS, _CAPACITY), f32),
            jax.ShapeDtypeStruct((_NUM_TOKENS, _NUM_EXPERTS), i32),
        ],
        compiler_params=pltpu.CompilerParams(
            dimension_semantics=("parallel",)),
    )(l1, l2, i1, i2, lc1, lc2, cnt)

    outm = pl.pallas_call(
        _phase3_body,
        grid=(_NB3,),
        in_specs=[pl.BlockSpec((_B3, _NUM_EXPERTS), lambda i: (i, 0))],
        out_specs=pl.BlockSpec((_B3, _NUM_EXPERTS, _CAPACITY),
                               lambda i: (i, 0, 0)),
        out_shape=jax.ShapeDtypeStruct(
            (_NUM_TOKENS, _NUM_EXPERTS, _CAPACITY), jnp.bool_),
        compiler_params=pltpu.CompilerParams(
            dimension_semantics=("parallel",)),
    )(pid)


    return laux[0, 0], outc, outm, cnt[0]


# speedup vs baseline: 1.1643x; 1.1643x over previous
import functools

import jax
import jax.numpy as jnp
from jax import lax
from jax.experimental import pallas as pl
from jax.experimental.pallas import tpu as pltpu

_NUM_TOKENS = 4096
_MODEL_DIM = 4096
_NUM_EXPERTS = 64
_CAPACITY = 128

_B1 = 1024
_NB1 = _NUM_TOKENS // _B1
_B2 = 256
_NB2 = _NUM_TOKENS // _B2
_NUM_ROWS = _NUM_TOKENS * _NUM_EXPERTS
_B3 = 4096
_NB3 = _NUM_ROWS // _B3

_GUMBEL_CACHE = None


def _gumbel_const():
    global _GUMBEL_CACHE
    if _GUMBEL_CACHE is None:
        _GUMBEL_CACHE = jax.random.gumbel(
            jax.random.key(42), (_NUM_TOKENS, _NUM_EXPERTS), jnp.float32)
    return _GUMBEL_CACHE


def _phase1_body(x_ref, wg_ref, gum_ref,
                 l1_ref, l2_ref, i1_ref, i2_ref, lc1_ref, lc2_ref,
                 cnt_ref, laux_ref,
                 accg_ref, c1_ref, c2_ref):
    i = pl.program_id(0)

    @pl.when(i == 0)
    def _init():
        accg_ref[...] = jnp.zeros_like(accg_ref)
        c1_ref[...] = jnp.zeros_like(c1_ref)
        c2_ref[...] = jnp.zeros_like(c2_ref)

    logits = jnp.dot(x_ref[...], wg_ref[...],
                     preferred_element_type=jnp.float32)
    iota_e = jax.lax.broadcasted_iota(jnp.int32, (_B1, _NUM_EXPERTS), 1)

    m1 = jnp.max(logits, axis=1, keepdims=True)
    idx1 = jnp.min(jnp.where(logits == m1, iota_e, _NUM_EXPERTS),
                   axis=1, keepdims=True)
    eq1 = iota_e == idx1

    e = jnp.exp(logits - m1)
    gates = e / jnp.sum(e, axis=1, keepdims=True)
    accg_ref[...] += jnp.sum(gates, axis=0, keepdims=True)

    lw = logits + gum_ref[...]
    lw1 = jnp.where(eq1, -jnp.inf, lw)
    m2 = jnp.max(lw1, axis=1, keepdims=True)
    idx2 = jnp.min(jnp.where(lw1 == m2, iota_e, _NUM_EXPERTS),
                   axis=1, keepdims=True)
    eq2 = iota_e == idx2
    l2v = jnp.sum(jnp.where(eq2, logits, 0.0), axis=1, keepdims=True)

    m1f = eq1.astype(jnp.float32)
    m2f = eq2.astype(jnp.float32)
    r = jax.lax.broadcasted_iota(jnp.int32, (_B1, _B1), 0)
    c = jax.lax.broadcasted_iota(jnp.int32, (_B1, _B1), 1)
    tri = (r >= c).astype(jnp.float32)
    cum1 = jnp.dot(tri, m1f, preferred_element_type=jnp.float32)
    cum2 = jnp.dot(tri, m2f, preferred_element_type=jnp.float32)

    c1p = c1_ref[...].astype(jnp.float32)
    c2p = c2_ref[...].astype(jnp.float32)
    loc1 = jnp.sum((c1p + cum1 - 1.0) * m1f, axis=1,
                   keepdims=True).astype(jnp.int32)
    loc2 = jnp.sum((c2p + cum2 - 1.0) * m2f, axis=1,
                   keepdims=True).astype(jnp.int32)

    c1_ref[...] += jnp.sum(eq1.astype(jnp.int32), axis=0, keepdims=True)
    c2_ref[...] += jnp.sum(eq2.astype(jnp.int32), axis=0, keepdims=True)

    l1_ref[...] = m1
    l2_ref[...] = l2v
    i1_ref[...] = idx1
    i2_ref[...] = idx2
    lc1_ref[...] = loc1
    lc2_ref[...] = loc2

    @pl.when(i == _NB1 - 1)
    def _finalize():
        cnt = c1_ref[...]
        cnt_ref[...] = cnt
        me = accg_ref[...] / _NUM_TOKENS
        ce = cnt.astype(jnp.float32) / _NUM_TOKENS
        laux_ref[...] = (jnp.sum(me * ce) * _NUM_EXPERTS).reshape(1, 1)


def _phase2_body(l1_ref, l2_ref, i1_ref, i2_ref, lc1_ref, lc2_ref, cnt_ref,
                 outc_ref, outm_ref):
    l1 = l1_ref[...]
    l2 = l2_ref[...]
    idx1 = i1_ref[...]
    idx2 = i2_ref[...]
    loc1 = lc1_ref[...]
    cnt = cnt_ref[...]

    iota_e = jax.lax.broadcasted_iota(jnp.int32, (_B2, _NUM_EXPERTS), 1)
    eq1 = iota_e == idx1
    eq2 = iota_e == idx2
    add2 = jnp.sum(jnp.where(eq2, cnt, 0), axis=1, keepdims=True)
    loc2 = lc2_ref[...] + add2

    k1 = (loc1 < _CAPACITY).astype(jnp.float32)
    k2 = (loc2 < _CAPACITY).astype(jnp.float32)
    b = jnp.exp(l2 - l1)
    denom = k1 + k2 * b
    inv = jnp.where(denom > 0, 1.0 / jnp.where(denom > 0, denom, 1.0), 0.0)
    w1 = k1 * inv
    w2 = k2 * b * inv

    iota_c = jax.lax.broadcasted_iota(jnp.int32, (_B2, _CAPACITY), 1)
    oh_c1 = (iota_c == loc1).astype(jnp.float32)
    oh_c2 = (iota_c == loc2).astype(jnp.float32)

    t1 = (w1 * eq1.astype(jnp.float32))[:, :, None] * oh_c1[:, None, :]
    t2 = (w2 * eq2.astype(jnp.float32))[:, :, None] * oh_c2[:, None, :]
    combine = t1 + t2
    outc_ref[...] = combine

    outm_ref[...] = combine != 0.0


def kernel(input, wg):
    gum = _gumbel_const()
    f32 = jnp.float32
    i32 = jnp.int32
    tok_spec = pl.BlockSpec((_B1, 1), lambda i: (i, 0))
    l1, l2, i1, i2, lc1, lc2, cnt, laux = pl.pallas_call(
        _phase1_body,
        grid=(_NB1,),
        in_specs=[
            pl.BlockSpec((_B1, _MODEL_DIM), lambda i: (i, 0)),
            pl.BlockSpec((_MODEL_DIM, _NUM_EXPERTS), lambda i: (0, 0)),
            pl.BlockSpec((_B1, _NUM_EXPERTS), lambda i: (i, 0)),
        ],
        out_specs=[
            tok_spec, tok_spec, tok_spec, tok_spec, tok_spec, tok_spec,
            pl.BlockSpec((1, _NUM_EXPERTS), lambda i: (0, 0)),
            pl.BlockSpec((1, 1), lambda i: (0, 0)),
        ],
        out_shape=[
            jax.ShapeDtypeStruct((_NUM_TOKENS, 1), f32),
            jax.ShapeDtypeStruct((_NUM_TOKENS, 1), f32),
            jax.ShapeDtypeStruct((_NUM_TOKENS, 1), i32),
            jax.ShapeDtypeStruct((_NUM_TOKENS, 1), i32),
            jax.ShapeDtypeStruct((_NUM_TOKENS, 1), i32),
            jax.ShapeDtypeStruct((_NUM_TOKENS, 1), i32),
            jax.ShapeDtypeStruct((1, _NUM_EXPERTS), i32),
            jax.ShapeDtypeStruct((1, 1), f32),
        ],
        scratch_shapes=[
            pltpu.VMEM((1, _NUM_EXPERTS), f32),
            pltpu.VMEM((1, _NUM_EXPERTS), i32),
            pltpu.VMEM((1, _NUM_EXPERTS), i32),
        ],
    )(input, wg, gum)

    tok2_spec = pl.BlockSpec((_B2, 1), lambda i: (i, 0))
    out3_spec = pl.BlockSpec((_B2, _NUM_EXPERTS, _CAPACITY),
                             lambda i: (i, 0, 0))
    outc, outm = pl.pallas_call(
        _phase2_body,
        grid=(_NB2,),
        in_specs=[
            tok2_spec, tok2_spec, tok2_spec, tok2_spec, tok2_spec, tok2_spec,
            pl.BlockSpec((1, _NUM_EXPERTS), lambda i: (0, 0)),
        ],
        out_specs=[out3_spec, out3_spec],
        out_shape=[
            jax.ShapeDtypeStruct((_NUM_TOKENS, _NUM_EXPERTS, _CAPACITY), f32),
            jax.ShapeDtypeStruct((_NUM_TOKENS, _NUM_EXPERTS, _CAPACITY),
                                 jnp.bool_),
        ],
        compiler_params=pltpu.CompilerParams(
            dimension_semantics=("parallel",)),
    )(l1, l2, i1, i2, lc1, lc2, cnt)


    return laux[0, 0], outc, outm, cnt[0]


# speedup vs baseline: 1.1893x; 1.0215x over previous
import jax
import jax.numpy as jnp
from jax.experimental import pallas as pl
from jax.experimental.pallas import tpu as pltpu

_NUM_TOKENS = 4096
_MODEL_DIM = 4096
_NUM_EXPERTS = 64
_CAPACITY = 128

_B1 = 512
_NB1 = _NUM_TOKENS // _B1
_B2 = 256
_NB2 = _NUM_TOKENS // _B2

_GUMBEL_CACHE = None


def _gumbel_const():
    global _GUMBEL_CACHE
    if _GUMBEL_CACHE is None:
        _GUMBEL_CACHE = jax.random.gumbel(
            jax.random.key(42), (_NUM_TOKENS, _NUM_EXPERTS), jnp.float32)
    return _GUMBEL_CACHE


def _phase1_body(x_ref, wg_ref, gum_ref,
                 l1_ref, l2_ref, i1_ref, i2_ref, lc1_ref, lc2_ref,
                 cnt_ref, laux_ref,
                 accg_ref, c1_ref, c2_ref):
    i = pl.program_id(0)

    @pl.when(i == 0)
    def _init():
        accg_ref[...] = jnp.zeros_like(accg_ref)
        c1_ref[...] = jnp.zeros_like(c1_ref)
        c2_ref[...] = jnp.zeros_like(c2_ref)

    logits = jnp.dot(x_ref[...], wg_ref[...],
                     preferred_element_type=jnp.float32)
    iota_e = jax.lax.broadcasted_iota(jnp.int32, (_B1, _NUM_EXPERTS), 1)

    m1 = jnp.max(logits, axis=1, keepdims=True)
    idx1 = jnp.min(jnp.where(logits == m1, iota_e, _NUM_EXPERTS),
                   axis=1, keepdims=True)
    eq1 = iota_e == idx1

    e = jnp.exp(logits - m1)
    gates = e / jnp.sum(e, axis=1, keepdims=True)
    accg_ref[...] += jnp.sum(gates, axis=0, keepdims=True)

    lw = logits + gum_ref[...]
    lw1 = jnp.where(eq1, -jnp.inf, lw)
    m2 = jnp.max(lw1, axis=1, keepdims=True)
    idx2 = jnp.min(jnp.where(lw1 == m2, iota_e, _NUM_EXPERTS),
                   axis=1, keepdims=True)
    eq2 = iota_e == idx2
    l2v = jnp.sum(jnp.where(eq2, logits, 0.0), axis=1, keepdims=True)

    m1f = eq1.astype(jnp.float32)
    m2f = eq2.astype(jnp.float32)
    r = jax.lax.broadcasted_iota(jnp.int32, (_B1, _B1), 0)
    c = jax.lax.broadcasted_iota(jnp.int32, (_B1, _B1), 1)
    tri = (r >= c).astype(jnp.float32)
    cum1 = jnp.dot(tri, m1f, preferred_element_type=jnp.float32)
    cum2 = jnp.dot(tri, m2f, preferred_element_type=jnp.float32)

    c1p = c1_ref[...].astype(jnp.float32)
    c2p = c2_ref[...].astype(jnp.float32)
    loc1 = jnp.sum((c1p + cum1 - 1.0) * m1f, axis=1,
                   keepdims=True).astype(jnp.int32)
    loc2 = jnp.sum((c2p + cum2 - 1.0) * m2f, axis=1,
                   keepdims=True).astype(jnp.int32)

    c1_ref[...] += jnp.sum(eq1.astype(jnp.int32), axis=0, keepdims=True)
    c2_ref[...] += jnp.sum(eq2.astype(jnp.int32), axis=0, keepdims=True)

    l1_ref[...] = m1
    l2_ref[...] = l2v
    i1_ref[...] = idx1
    i2_ref[...] = idx2
    lc1_ref[...] = loc1
    lc2_ref[...] = loc2

    @pl.when(i == _NB1 - 1)
    def _finalize():
        cnt = c1_ref[...]
        cnt_ref[...] = cnt
        me = accg_ref[...] / _NUM_TOKENS
        ce = cnt.astype(jnp.float32) / _NUM_TOKENS
        laux_ref[...] = (jnp.sum(me * ce) * _NUM_EXPERTS).reshape(1, 1)


def _phase2_body(l1_ref, l2_ref, i1_ref, i2_ref, lc1_ref, lc2_ref, cnt_ref,
                 outc_ref, outm_ref):
    l1 = l1_ref[...]
    l2 = l2_ref[...]
    idx1 = i1_ref[...]
    idx2 = i2_ref[...]
    loc1 = lc1_ref[...]
    cnt = cnt_ref[...]

    iota_e = jax.lax.broadcasted_iota(jnp.int32, (_B2, _NUM_EXPERTS), 1)
    eq1 = iota_e == idx1
    eq2 = iota_e == idx2
    add2 = jnp.sum(jnp.where(eq2, cnt, 0), axis=1, keepdims=True)
    loc2 = lc2_ref[...] + add2

    k1 = (loc1 < _CAPACITY).astype(jnp.float32)
    k2 = (loc2 < _CAPACITY).astype(jnp.float32)
    b = jnp.exp(l2 - l1)
    denom = k1 + k2 * b
    inv = jnp.where(denom > 0, 1.0 / jnp.where(denom > 0, denom, 1.0), 0.0)
    w1 = k1 * inv
    w2 = k2 * b * inv

    iota_c = jax.lax.broadcasted_iota(jnp.int32, (_B2, _CAPACITY), 1)
    oh_c1 = (iota_c == loc1).astype(jnp.float32)
    oh_c2 = (iota_c == loc2).astype(jnp.float32)

    t1 = (w1 * eq1.astype(jnp.float32))[:, :, None] * oh_c1[:, None, :]
    t2 = (w2 * eq2.astype(jnp.float32))[:, :, None] * oh_c2[:, None, :]
    combine = t1 + t2
    outc_ref[...] = combine
    outm_ref[...] = combine != 0.0


def kernel(input, wg):
    gum = _gumbel_const()
    f32 = jnp.float32
    i32 = jnp.int32
    tok_spec = pl.BlockSpec((_B1, 1), lambda i: (i, 0))
    l1, l2, i1, i2, lc1, lc2, cnt, laux = pl.pallas_call(
        _phase1_body,
        grid=(_NB1,),
        in_specs=[
            pl.BlockSpec((_B1, _MODEL_DIM), lambda i: (i, 0)),
            pl.BlockSpec((_MODEL_DIM, _NUM_EXPERTS), lambda i: (0, 0)),
            pl.BlockSpec((_B1, _NUM_EXPERTS), lambda i: (i, 0)),
        ],
        out_specs=[
            tok_spec, tok_spec, tok_spec, tok_spec, tok_spec, tok_spec,
            pl.BlockSpec((1, _NUM_EXPERTS), lambda i: (0, 0)),
            pl.BlockSpec((1, 1), lambda i: (0, 0)),
        ],
        out_shape=[
            jax.ShapeDtypeStruct((_NUM_TOKENS, 1), f32),
            jax.ShapeDtypeStruct((_NUM_TOKENS, 1), f32),
            jax.ShapeDtypeStruct((_NUM_TOKENS, 1), i32),
            jax.ShapeDtypeStruct((_NUM_TOKENS, 1), i32),
            jax.ShapeDtypeStruct((_NUM_TOKENS, 1), i32),
            jax.ShapeDtypeStruct((_NUM_TOKENS, 1), i32),
            jax.ShapeDtypeStruct((1, _NUM_EXPERTS), i32),
            jax.ShapeDtypeStruct((1, 1), f32),
        ],
        scratch_shapes=[
            pltpu.VMEM((1, _NUM_EXPERTS), f32),
            pltpu.VMEM((1, _NUM_EXPERTS), i32),
            pltpu.VMEM((1, _NUM_EXPERTS), i32),
        ],
    )(input, wg, gum)

    tok2_spec = pl.BlockSpec((_B2, 1), lambda i: (i, 0))
    out3_spec = pl.BlockSpec((_B2, _NUM_EXPERTS, _CAPACITY),
                             lambda i: (i, 0, 0))
    outc, outm = pl.pallas_call(
        _phase2_body,
        grid=(_NB2,),
        in_specs=[
            tok2_spec, tok2_spec, tok2_spec, tok2_spec, tok2_spec, tok2_spec,
            pl.BlockSpec((1, _NUM_EXPERTS), lambda i: (0, 0)),
        ],
        out_specs=[out3_spec, out3_spec],
        out_shape=[
            jax.ShapeDtypeStruct((_NUM_TOKENS, _NUM_EXPERTS, _CAPACITY), f32),
            jax.ShapeDtypeStruct((_NUM_TOKENS, _NUM_EXPERTS, _CAPACITY),
                                 jnp.bool_),
        ],
        compiler_params=pltpu.CompilerParams(
            dimension_semantics=("parallel",)),
    )(l1, l2, i1, i2, lc1, lc2, cnt)

    return laux[0, 0], outc, outm, cnt[0]
